# baseline (speedup 1.0000x reference)
import functools

import jax
import jax.numpy as jnp
from jax import lax
from jax.experimental import pallas as pl
from jax.experimental.pallas import tpu as pltpu
from jax.experimental.pallas import tpu_sc as plsc

B = 4096
D = 64
LEN_VOCAB = 51
OUT_W = 2 * D + LEN_VOCAB + D

_NC = 2
_NS = 16
_NW = _NC * _NS
_BPW = B // _NW

_mesh = plsc.VectorSubcoreMesh(core_axis_name="c", subcore_axis_name="s")


@functools.partial(
    pl.kernel,
    mesh=_mesh,
    out_type=[
        jax.ShapeDtypeStruct((B, D), jnp.float32),
        jax.ShapeDtypeStruct((B, D), jnp.float32),
        jax.ShapeDtypeStruct((B, D), jnp.float32),
    ],
    scratch_types=[
        pltpu.VMEM((_BPW,), jnp.int32),
        pltpu.VMEM((_BPW,), jnp.int32),
        pltpu.VMEM((_BPW,), jnp.int32),
        pltpu.VMEM((_BPW, D), jnp.float32),
        pltpu.VMEM((_BPW, D), jnp.float32),
        pltpu.VMEM((_BPW, D), jnp.float32),
        pltpu.SemaphoreType.DMA,
        pltpu.SemaphoreType.DMA,
    ],
)
def _gather3(adv_t, brd_t, ind_t, ia, ib, ii, oa, ob, oi,
             va, vb, vi, ra, rb, ri, sem_idx, sem):
    wid = lax.axis_index("s") * _NC + lax.axis_index("c")
    base = wid * _BPW
    ca = pltpu.async_copy(ia.at[pl.ds(base, _BPW)], va, sem_idx)
    cb = pltpu.async_copy(ib.at[pl.ds(base, _BPW)], vb, sem_idx)
    ci = pltpu.async_copy(ii.at[pl.ds(base, _BPW)], vi, sem_idx)
    ca.wait()
    cb.wait()
    ci.wait()

    @pl.loop(0, _BPW, step=16)
    def _(r0):
        idxa = va[pl.ds(r0, 16)]
        idxb = vb[pl.ds(r0, 16)]
        idxi = vi[pl.ds(r0, 16)]
        for j in range(16):
            r = r0 + j
            pltpu.async_copy(adv_t.at[pl.ds(idxa[j], 1), :],
                             ra.at[pl.ds(r, 1), :], sem)
            pltpu.async_copy(brd_t.at[pl.ds(idxb[j], 1), :],
                             rb.at[pl.ds(r, 1), :], sem)
            pltpu.async_copy(ind_t.at[pl.ds(idxi[j], 1), :],
                             ri.at[pl.ds(r, 1), :], sem)

    @pl.loop(0, _BPW)
    def _(r):
        pltpu.make_async_copy(adv_t.at[pl.ds(0, 1), :],
                              ra.at[pl.ds(0, 1), :], sem).wait()
        pltpu.make_async_copy(brd_t.at[pl.ds(0, 1), :],
                              rb.at[pl.ds(0, 1), :], sem).wait()
        pltpu.make_async_copy(ind_t.at[pl.ds(0, 1), :],
                              ri.at[pl.ds(0, 1), :], sem).wait()

    pltpu.sync_copy(ra, oa.at[pl.ds(base, _BPW), :])
    pltpu.sync_copy(rb, ob.at[pl.ds(base, _BPW), :])
    pltpu.sync_copy(ri, oi.at[pl.ds(base, _BPW), :])


_TBLK = 1024


def _transpose_body(in_ref, o_ref):
    o_ref[...] = in_ref[...].T


def _transpose(table_t):
    v = table_t.shape[1]
    return pl.pallas_call(
        _transpose_body,
        grid=(pl.cdiv(v, _TBLK),),
        in_specs=[pl.BlockSpec((D, _TBLK), lambda i: (0, i))],
        out_specs=pl.BlockSpec((_TBLK, D), lambda i: (i, 0)),
        out_shape=jax.ShapeDtypeStruct((v, D), jnp.float32),
    )(table_t)


_BLK = 512


def _assemble_body(c_ref, a_ref, b_ref, i_ref, o_ref):
    oh = (c_ref[...] == lax.broadcasted_iota(jnp.int32, (_BLK, LEN_VOCAB), 1))
    o_ref[...] = jnp.concatenate(
        [a_ref[...], b_ref[...], oh.astype(jnp.float32), i_ref[...]], axis=1)


def _assemble(cl2, adv_emb, brd_emb, ind_emb):
    return pl.pallas_call(
        _assemble_body,
        grid=(B // _BLK,),
        in_specs=[
            pl.BlockSpec((_BLK, 1), lambda i: (i, 0)),
            pl.BlockSpec((_BLK, D), lambda i: (i, 0)),
            pl.BlockSpec((_BLK, D), lambda i: (i, 0)),
            pl.BlockSpec((_BLK, D), lambda i: (i, 0)),
        ],
        out_specs=pl.BlockSpec((_BLK, OUT_W), lambda i: (i, 0)),
        out_shape=jax.ShapeDtypeStruct((B, OUT_W), jnp.float32),
    )(cl2, adv_emb, brd_emb, ind_emb)


def kernel(advertiser_id, brand_id, industry, campaign_length,
           adv_table, brand_table, ind_table):
    adv_lin = _transpose(adv_table.T)
    brd_lin = _transpose(brand_table.T)
    adv_emb, brd_emb, ind_emb = _gather3(
        adv_lin, brd_lin, ind_table,
        advertiser_id, brand_id, industry)
    return _assemble(campaign_length.reshape(B, 1), adv_emb, brd_emb, ind_emb)


# speedup vs baseline: 1.7630x; 1.7630x over previous
import functools

import jax
import jax.numpy as jnp
from jax import lax
from jax.experimental import pallas as pl
from jax.experimental.pallas import tpu as pltpu
from jax.experimental.pallas import tpu_sc as plsc

B = 4096
D = 64
LEN_VOCAB = 51
OUT_W = 2 * D + LEN_VOCAB + D

_NC = 2
_NS = 16
_NW = _NC * _NS
_BPW = B // _NW

_mesh = plsc.VectorSubcoreMesh(core_axis_name="c", subcore_axis_name="s")


@functools.partial(
    pl.kernel,
    mesh=_mesh,
    out_type=[
        jax.ShapeDtypeStruct((B, D), jnp.float32),
        jax.ShapeDtypeStruct((B, D), jnp.float32),
        jax.ShapeDtypeStruct((B, D), jnp.float32),
    ],
    scratch_types=[
        pltpu.VMEM((_BPW,), jnp.int32),
        pltpu.VMEM((_BPW,), jnp.int32),
        pltpu.VMEM((_BPW,), jnp.int32),
        pltpu.VMEM((_BPW, D), jnp.float32),
        pltpu.VMEM((_BPW, D), jnp.float32),
        pltpu.VMEM((_BPW, D), jnp.float32),
        pltpu.SemaphoreType.DMA,
        pltpu.SemaphoreType.DMA,
    ],
)
def _gather3(adv_t, brd_t, ind_t, ia, ib, ii, oa, ob, oi,
             va, vb, vi, ra, rb, ri, sem_idx, sem):
    wid = lax.axis_index("s") * _NC + lax.axis_index("c")
    base = wid * _BPW
    ca = pltpu.async_copy(ia.at[pl.ds(base, _BPW)], va, sem_idx)
    cb = pltpu.async_copy(ib.at[pl.ds(base, _BPW)], vb, sem_idx)
    ci = pltpu.async_copy(ii.at[pl.ds(base, _BPW)], vi, sem_idx)
    ca.wait()
    cb.wait()
    ci.wait()

    @pl.loop(0, _BPW, step=16)
    def _(r0):
        idxa = va[pl.ds(r0, 16)]
        idxb = vb[pl.ds(r0, 16)]
        idxi = vi[pl.ds(r0, 16)]
        for j in range(16):
            r = r0 + j
            pltpu.async_copy(adv_t.at[pl.ds(idxa[j], 1), :],
                             ra.at[pl.ds(r, 1), :], sem)
            pltpu.async_copy(brd_t.at[pl.ds(idxb[j], 1), :],
                             rb.at[pl.ds(r, 1), :], sem)
            pltpu.async_copy(ind_t.at[pl.ds(idxi[j], 1), :],
                             ri.at[pl.ds(r, 1), :], sem)

    @pl.loop(0, _BPW)
    def _(r):
        pltpu.make_async_copy(adv_t.at[pl.ds(0, 1), :],
                              ra.at[pl.ds(0, 1), :], sem).wait()
        pltpu.make_async_copy(brd_t.at[pl.ds(0, 1), :],
                              rb.at[pl.ds(0, 1), :], sem).wait()
        pltpu.make_async_copy(ind_t.at[pl.ds(0, 1), :],
                              ri.at[pl.ds(0, 1), :], sem).wait()

    pltpu.sync_copy(ra, oa.at[pl.ds(base, _BPW), :])
    pltpu.sync_copy(rb, ob.at[pl.ds(base, _BPW), :])
    pltpu.sync_copy(ri, oi.at[pl.ds(base, _BPW), :])


_BLK = 512


def _assemble_body(c_ref, a_ref, b_ref, i_ref, o_ref):
    oh = (c_ref[...] ==
          lax.broadcasted_iota(jnp.int32, (LEN_VOCAB, _BLK), 0))
    o_ref[pl.ds(0, D), :] = a_ref[...].T
    o_ref[pl.ds(D, D), :] = b_ref[...].T
    o_ref[pl.ds(2 * D, LEN_VOCAB), :] = oh.astype(jnp.float32)
    o_ref[pl.ds(2 * D + LEN_VOCAB, D), :] = i_ref[...].T


def _assemble(cl2, adv_emb, brd_emb, ind_emb):
    return pl.pallas_call(
        _assemble_body,
        grid=(B // _BLK,),
        in_specs=[
            pl.BlockSpec((1, _BLK), lambda i: (0, i)),
            pl.BlockSpec((_BLK, D), lambda i: (i, 0)),
            pl.BlockSpec((_BLK, D), lambda i: (i, 0)),
            pl.BlockSpec((_BLK, D), lambda i: (i, 0)),
        ],
        out_specs=pl.BlockSpec((OUT_W, _BLK), lambda i: (0, i)),
        out_shape=jax.ShapeDtypeStruct((OUT_W, B), jnp.float32),
    )(cl2, adv_emb, brd_emb, ind_emb)


def kernel(advertiser_id, brand_id, industry, campaign_length,
           adv_table, brand_table, ind_table):
    adv_emb, brd_emb, ind_emb = _gather3(
        adv_table, brand_table, ind_table,
        advertiser_id, brand_id, industry)
    out_t = _assemble(campaign_length.reshape(1, B),
                      adv_emb, brd_emb, ind_emb)
    return out_t.T


# speedup vs baseline: 1.8007x; 1.0214x over previous
import functools

import jax
import jax.numpy as jnp
from jax import lax
from jax.experimental import pallas as pl
from jax.experimental.pallas import tpu as pltpu
from jax.experimental.pallas import tpu_sc as plsc

B = 4096
D = 64
LEN_VOCAB = 51
OUT_W = 2 * D + LEN_VOCAB + D

_NC = 2
_NS = 16
_NW = _NC * _NS
_BPW = B // _NW

_mesh = plsc.VectorSubcoreMesh(core_axis_name="c", subcore_axis_name="s")


@functools.partial(
    pl.kernel,
    mesh=_mesh,
    out_type=[
        jax.ShapeDtypeStruct((B, D), jnp.float32),
        jax.ShapeDtypeStruct((B, D), jnp.float32),
        jax.ShapeDtypeStruct((B, D), jnp.float32),
    ],
    scratch_types=[
        pltpu.VMEM((_BPW,), jnp.int32),
        pltpu.VMEM((_BPW,), jnp.int32),
        pltpu.VMEM((_BPW,), jnp.int32),
        pltpu.VMEM((_BPW, D), jnp.float32),
        pltpu.VMEM((_BPW, D), jnp.float32),
        pltpu.VMEM((_BPW, D), jnp.float32),
        pltpu.SemaphoreType.DMA,
        pltpu.SemaphoreType.DMA,
    ],
)
def _gather3(adv_t, brd_t, ind_t, ia, ib, ii, oa, ob, oi,
             va, vb, vi, ra, rb, ri, sem_idx, sem):
    wid = lax.axis_index("s") * _NC + lax.axis_index("c")
    base = wid * _BPW
    ca = pltpu.async_copy(ia.at[pl.ds(base, _BPW)], va, sem_idx)
    cb = pltpu.async_copy(ib.at[pl.ds(base, _BPW)], vb, sem_idx)
    ci = pltpu.async_copy(ii.at[pl.ds(base, _BPW)], vi, sem_idx)
    ca.wait()
    cb.wait()
    ci.wait()

    @pl.loop(0, _BPW, step=16)
    def _(r0):
        idxa = va[pl.ds(r0, 16)]
        idxb = vb[pl.ds(r0, 16)]
        idxi = vi[pl.ds(r0, 16)]
        for j in range(16):
            r = r0 + j
            pltpu.async_copy(adv_t.at[pl.ds(idxa[j], 1), :],
                             ra.at[pl.ds(r, 1), :], sem)
            pltpu.async_copy(brd_t.at[pl.ds(idxb[j], 1), :],
                             rb.at[pl.ds(r, 1), :], sem)
            pltpu.async_copy(ind_t.at[pl.ds(idxi[j], 1), :],
                             ri.at[pl.ds(r, 1), :], sem)

    @pl.loop(0, _BPW)
    def _(r):
        pltpu.make_async_copy(adv_t.at[pl.ds(0, 1), :],
                              ra.at[pl.ds(0, 1), :], sem).wait()
        pltpu.make_async_copy(brd_t.at[pl.ds(0, 1), :],
                              rb.at[pl.ds(0, 1), :], sem).wait()
        pltpu.make_async_copy(ind_t.at[pl.ds(0, 1), :],
                              ri.at[pl.ds(0, 1), :], sem).wait()

    pltpu.sync_copy(ra, oa.at[pl.ds(base, _BPW), :])
    pltpu.sync_copy(rb, ob.at[pl.ds(base, _BPW), :])
    pltpu.sync_copy(ri, oi.at[pl.ds(base, _BPW), :])


_BLK = 1024


def _assemble_body(c_ref, a_ref, b_ref, i_ref, o_ref):
    oh = (c_ref[...] ==
          lax.broadcasted_iota(jnp.int32, (LEN_VOCAB, _BLK), 0))
    o_ref[pl.ds(0, D), :] = a_ref[...].T
    o_ref[pl.ds(D, D), :] = b_ref[...].T
    o_ref[pl.ds(2 * D, LEN_VOCAB), :] = oh.astype(jnp.float32)
    o_ref[pl.ds(2 * D + LEN_VOCAB, D), :] = i_ref[...].T


def _assemble(cl2, adv_emb, brd_emb, ind_emb):
    return pl.pallas_call(
        _assemble_body,
        grid=(B // _BLK,),
        in_specs=[
            pl.BlockSpec((1, _BLK), lambda i: (0, i)),
            pl.BlockSpec((_BLK, D), lambda i: (i, 0)),
            pl.BlockSpec((_BLK, D), lambda i: (i, 0)),
            pl.BlockSpec((_BLK, D), lambda i: (i, 0)),
        ],
        out_specs=pl.BlockSpec((OUT_W, _BLK), lambda i: (0, i)),
        out_shape=jax.ShapeDtypeStruct((OUT_W, B), jnp.float32),
    )(cl2, adv_emb, brd_emb, ind_emb)


def kernel(advertiser_id, brand_id, industry, campaign_length,
           adv_table, brand_table, ind_table):
    adv_emb, brd_emb, ind_emb = _gather3(
        adv_table, brand_table, ind_table,
        advertiser_id, brand_id, industry)
    out_t = _assemble(campaign_length.reshape(1, B),
                      adv_emb, brd_emb, ind_emb)
    return out_t.T


# speedup vs baseline: 1.8119x; 1.0062x over previous
import functools

import jax
import jax.numpy as jnp
from jax import lax
from jax.experimental import pallas as pl
from jax.experimental.pallas import tpu as pltpu
from jax.experimental.pallas import tpu_sc as plsc

B = 4096
D = 64
LEN_VOCAB = 51
OUT_W = 2 * D + LEN_VOCAB + D

_NC = 2
_NS = 16
_NW = _NC * _NS
_BPW = B // _NW

_mesh = plsc.VectorSubcoreMesh(core_axis_name="c", subcore_axis_name="s")


@functools.partial(
    pl.kernel,
    mesh=_mesh,
    out_type=[
        jax.ShapeDtypeStruct((B, D), jnp.float32),
        jax.ShapeDtypeStruct((B, D), jnp.float32),
        jax.ShapeDtypeStruct((B, D), jnp.float32),
    ],
    scratch_types=[
        pltpu.VMEM((_BPW,), jnp.int32),
        pltpu.VMEM((_BPW,), jnp.int32),
        pltpu.VMEM((_BPW,), jnp.int32),
        pltpu.VMEM((_BPW, D), jnp.float32),
        pltpu.VMEM((_BPW, D), jnp.float32),
        pltpu.VMEM((_BPW, D), jnp.float32),
        pltpu.SemaphoreType.DMA,
        pltpu.SemaphoreType.DMA,
    ],
)
def _gather3(adv_t, brd_t, ind_t, ia, ib, ii, oa, ob, oi,
             va, vb, vi, ra, rb, ri, sem_idx, sem):
    wid = lax.axis_index("s") * _NC + lax.axis_index("c")
    base = wid * _BPW
    ca = pltpu.async_copy(ia.at[pl.ds(base, _BPW)], va, sem_idx)
    cb = pltpu.async_copy(ib.at[pl.ds(base, _BPW)], vb, sem_idx)
    ci = pltpu.async_copy(ii.at[pl.ds(base, _BPW)], vi, sem_idx)
    ca.wait()
    cb.wait()
    ci.wait()

    @pl.loop(0, _BPW, step=16)
    def _(r0):
        idxa = va[pl.ds(r0, 16)]
        idxb = vb[pl.ds(r0, 16)]
        idxi = vi[pl.ds(r0, 16)]
        for j in range(16):
            r = r0 + j
            pltpu.async_copy(adv_t.at[pl.ds(idxa[j], 1), :],
                             ra.at[pl.ds(r, 1), :], sem)
            pltpu.async_copy(brd_t.at[pl.ds(idxb[j], 1), :],
                             rb.at[pl.ds(r, 1), :], sem)
            pltpu.async_copy(ind_t.at[pl.ds(idxi[j], 1), :],
                             ri.at[pl.ds(r, 1), :], sem)

    @pl.loop(0, _BPW)
    def _(r):
        pltpu.make_async_copy(adv_t.at[pl.ds(0, 1), :],
                              ra.at[pl.ds(0, 1), :], sem).wait()
        pltpu.make_async_copy(brd_t.at[pl.ds(0, 1), :],
                              rb.at[pl.ds(0, 1), :], sem).wait()
        pltpu.make_async_copy(ind_t.at[pl.ds(0, 1), :],
                              ri.at[pl.ds(0, 1), :], sem).wait()

    pltpu.sync_copy(ra, oa.at[pl.ds(base, _BPW), :])
    pltpu.sync_copy(rb, ob.at[pl.ds(base, _BPW), :])
    pltpu.sync_copy(ri, oi.at[pl.ds(base, _BPW), :])


_BLK = 2048


def _assemble_body(c_ref, a_ref, b_ref, i_ref, o_ref):
    oh = (c_ref[...] ==
          lax.broadcasted_iota(jnp.int32, (LEN_VOCAB, _BLK), 0))
    o_ref[pl.ds(0, D), :] = a_ref[...].T
    o_ref[pl.ds(D, D), :] = b_ref[...].T
    o_ref[pl.ds(2 * D, LEN_VOCAB), :] = oh.astype(jnp.float32)
    o_ref[pl.ds(2 * D + LEN_VOCAB, D), :] = i_ref[...].T


def _assemble(cl2, adv_emb, brd_emb, ind_emb):
    return pl.pallas_call(
        _assemble_body,
        grid=(B // _BLK,),
        in_specs=[
            pl.BlockSpec((1, _BLK), lambda i: (0, i)),
            pl.BlockSpec((_BLK, D), lambda i: (i, 0)),
            pl.BlockSpec((_BLK, D), lambda i: (i, 0)),
            pl.BlockSpec((_BLK, D), lambda i: (i, 0)),
        ],
        out_specs=pl.BlockSpec((OUT_W, _BLK), lambda i: (0, i)),
        out_shape=jax.ShapeDtypeStruct((OUT_W, B), jnp.float32),
    )(cl2, adv_emb, brd_emb, ind_emb)


def kernel(advertiser_id, brand_id, industry, campaign_length,
           adv_table, brand_table, ind_table):
    adv_emb, brd_emb, ind_emb = _gather3(
        adv_table, brand_table, ind_table,
        advertiser_id, brand_id, industry)
    out_t = _assemble(campaign_length.reshape(1, B),
                      adv_emb, brd_emb, ind_emb)
    return out_t.T
